# Initial kernel scaffold; baseline (speedup 1.0000x reference)
import jax
import jax.numpy as jnp
from jax.experimental import pallas as pl


def kernel(x_user, x_item, edge_index_ubi, edge_index_ibu, Wl1_ubi, bl1_ubi, Wr1_ubi, Wl1_ibu, bl1_ibu, Wr1_ibu, Wl2_ubi, bl2_ubi, Wr2_ubi, Wl2_ibu, bl2_ibu, Wr2_ibu, Wres_user, bres_user, Wres_item, bres_item):
    raise NotImplementedError("write your pallas kernel here")


# speedup vs baseline: 6.1267x; 6.1267x over previous
import functools

import jax
import jax.numpy as jnp
from jax import lax
from jax.experimental import pallas as pl
from jax.experimental.pallas import tpu as pltpu
from jax.experimental.pallas import tpu_sc as plsc

N = 10000
NPAD = 10240
E = 320000
D = 128

NS = 16
EPT = E // NS
CHUNK = 80
NCHUNK = EPT // CHUNK
NCHUNK2 = NCHUNK // 2
RPT = NPAD // NS
ZROWS = 128


def _zero_fill(ref, rows, cols):
    zv = jnp.zeros((16,), jnp.float32)

    @pl.loop(0, rows)
    def _(i):
        for c in range(cols // 16):
            ref[i, pl.ds(c * 16, 16)] = zv


def _fill_ones(ref, rows, cols):
    ov = jnp.ones((16,), jnp.float32)

    @pl.loop(0, rows)
    def _(i):
        for c in range(cols // 16):
            ref[i, pl.ds(c * 16, 16)] = ov


def _make_seg_kernel(with_init):
    mesh = plsc.VectorSubcoreMesh(core_axis_name="c", subcore_axis_name="s",
                                  num_cores=2, num_subcores=NS)

    out_type = [jax.ShapeDtypeStruct((2, NPAD, D), jnp.float32)]
    scratch = [
        pltpu.VMEM((NCHUNK2, CHUNK), jnp.int32),
        pltpu.VMEM((NCHUNK2, CHUNK), jnp.int32),
        pltpu.VMEM((CHUNK, D), jnp.float32),
        pltpu.VMEM((ZROWS, D), jnp.float32),
        pltpu.VMEM_SHARED((NPAD, D), jnp.float32),
    ]

    def body(tab, src, dst, *rest):
        if with_init:
            init, agg_out, src_v, dst_v, rows_v, zrow_v, acc_sh = rest
        else:
            agg_out, src_v, dst_v, rows_v, zrow_v, acc_sh = rest

        cid = lax.axis_index("c")
        sid = lax.axis_index("s")

        if with_init:
            pltpu.sync_copy(init.at[cid, pl.ds(sid * RPT, RPT)],
                            acc_sh.at[pl.ds(sid * RPT, RPT)])
        else:
            _zero_fill(zrow_v, ZROWS, D)
            for k in range(RPT // ZROWS):
                pltpu.sync_copy(
                    zrow_v, acc_sh.at[pl.ds(sid * RPT + k * ZROWS, ZROWS)])
        plsc.subcore_barrier()

        pltpu.sync_copy(src.at[cid, sid], src_v)
        pltpu.sync_copy(dst.at[cid, sid], dst_v)
        table = tab.at[cid]

        @pl.loop(0, NCHUNK2)
        def _(j):
            pltpu.sync_copy(table.at[src_v.at[j]], rows_v)
            pltpu.sync_copy(rows_v, acc_sh.at[dst_v.at[j]], add=True)

        plsc.subcore_barrier()
        pltpu.sync_copy(acc_sh.at[pl.ds(sid * RPT, RPT)],
                        agg_out.at[cid, pl.ds(sid * RPT, RPT)])

    return pl.kernel(
        body, out_type=out_type, mesh=mesh, scratch_types=scratch,
        compiler_params=pltpu.CompilerParams(use_tc_tiling_on_sc=False),
        name="seg_sum_acc" if with_init else "seg_sum")


def _make_cnt_kernel():
    mesh = plsc.VectorSubcoreMesh(core_axis_name="c", subcore_axis_name="s",
                                  num_cores=2, num_subcores=NS)

    def body(dst, cnt_out, dst_v, ones_v, zcnt_v, cntacc_sh):
        cid = lax.axis_index("c")
        sid = lax.axis_index("s")

        _fill_ones(ones_v, CHUNK, 16)
        _zero_fill(zcnt_v, RPT, 16)
        pltpu.sync_copy(zcnt_v, cntacc_sh.at[pl.ds(sid * RPT, RPT)])
        plsc.subcore_barrier()

        pltpu.sync_copy(dst.at[cid, sid], dst_v)

        @pl.loop(0, NCHUNK)
        def _(j):
            pltpu.sync_copy(ones_v, cntacc_sh.at[dst_v.at[j]], add=True)

        plsc.subcore_barrier()
        pltpu.sync_copy(cntacc_sh.at[pl.ds(sid * RPT, RPT)],
                        cnt_out.at[cid, pl.ds(sid * RPT, RPT)])

    return pl.kernel(
        body,
        out_type=[jax.ShapeDtypeStruct((2, NPAD, 16), jnp.float32)],
        mesh=mesh,
        scratch_types=[
            pltpu.VMEM((NCHUNK, CHUNK), jnp.int32),
            pltpu.VMEM((CHUNK, 16), jnp.float32),
            pltpu.VMEM((RPT, 16), jnp.float32),
            pltpu.VMEM_SHARED((NPAD, 16), jnp.float32),
        ],
        compiler_params=pltpu.CompilerParams(use_tc_tiling_on_sc=False),
        name="seg_cnt")


_make_seg_kernel = functools.lru_cache(maxsize=None)(_make_seg_kernel)
_make_cnt_kernel = functools.lru_cache(maxsize=None)(_make_cnt_kernel)


def _seg_sum(tab, src, dst):
    (part,) = _make_seg_kernel(False)(tab, src[:, :, 0], dst[:, :, 0])
    (agg,) = _make_seg_kernel(True)(tab, src[:, :, 1], dst[:, :, 1], part)
    return agg


def _seg_counts(dst):
    (cnt,) = _make_cnt_kernel()(dst)
    return cnt


BLK = 1024


def _layer1_body(agg, cnt, x, Wl1, bl1, Wr1, Wl2x, Wr2x, Wres,
                 p, r2, rres):
    c = jnp.maximum(cnt[:, 0:1], 1.0)
    mean = agg[:] / c
    dn = (((1,), (1,)), ((), ()))
    t = (lax.dot_general(mean, Wl1[:], dn, preferred_element_type=jnp.float32)
         + bl1[:]
         + lax.dot_general(x[:], Wr1[:], dn,
                           preferred_element_type=jnp.float32))
    nrm = jnp.maximum(jnp.sqrt(jnp.sum(t * t, axis=1, keepdims=True)), 1e-12)
    x1 = jnp.maximum(t / nrm, 0.0)
    p[:] = lax.dot_general(x1, Wl2x[:], dn, preferred_element_type=jnp.float32)
    r2[:] = lax.dot_general(x1, Wr2x[:], dn,
                            preferred_element_type=jnp.float32)
    rres[:] = lax.dot_general(x1, Wres[:], dn,
                              preferred_element_type=jnp.float32)


def _layer1(agg, cnt, x, Wl1, bl1, Wr1, Wl2x, Wr2x, Wres):
    grid = NPAD // BLK
    return pl.pallas_call(
        _layer1_body,
        grid=grid,
        in_specs=[
            pl.BlockSpec((BLK, D), lambda i: (i, 0)),
            pl.BlockSpec((BLK, 16), lambda i: (i, 0)),
            pl.BlockSpec((BLK, D), lambda i: (i, 0)),
            pl.BlockSpec((2 * D, D), lambda i: (0, 0)),
            pl.BlockSpec((1, 2 * D), lambda i: (0, 0)),
            pl.BlockSpec((2 * D, D), lambda i: (0, 0)),
            pl.BlockSpec((D, 2 * D), lambda i: (0, 0)),
            pl.BlockSpec((D, 2 * D), lambda i: (0, 0)),
            pl.BlockSpec((D, 2 * D), lambda i: (0, 0)),
        ],
        out_specs=[
            pl.BlockSpec((BLK, D), lambda i: (i, 0)),
            pl.BlockSpec((BLK, D), lambda i: (i, 0)),
            pl.BlockSpec((BLK, D), lambda i: (i, 0)),
        ],
        out_shape=[jax.ShapeDtypeStruct((NPAD, D), jnp.float32)] * 3,
        name="sage_layer1",
    )(agg, cnt, x, Wl1, bl1.reshape(1, -1), Wr1, Wl2x, Wr2x, Wres)


def _layer2_body(agg2, cnt, r2, rres, bl2, bres, out):
    c = jnp.maximum(cnt[:, 0:1], 1.0)
    t = agg2[:] / c + bl2[:] + r2[:]
    nrm = jnp.maximum(jnp.sqrt(jnp.sum(t * t, axis=1, keepdims=True)), 1e-12)
    out[:] = t / nrm + rres[:] + bres[:]


def _layer2(agg2, cnt, r2, rres, bl2, bres):
    grid = NPAD // BLK
    return pl.pallas_call(
        _layer2_body,
        grid=grid,
        in_specs=[
            pl.BlockSpec((BLK, D), lambda i: (i, 0)),
            pl.BlockSpec((BLK, 16), lambda i: (i, 0)),
            pl.BlockSpec((BLK, D), lambda i: (i, 0)),
            pl.BlockSpec((BLK, D), lambda i: (i, 0)),
            pl.BlockSpec((1, D), lambda i: (0, 0)),
            pl.BlockSpec((1, D), lambda i: (0, 0)),
        ],
        out_specs=pl.BlockSpec((BLK, D), lambda i: (i, 0)),
        out_shape=jax.ShapeDtypeStruct((NPAD, D), jnp.float32),
        name="sage_layer2",
    )(agg2, cnt, r2, rres, bl2.reshape(1, -1), bres.reshape(1, -1))


def kernel(x_user, x_item, edge_index_ubi, edge_index_ibu,
           Wl1_ubi, bl1_ubi, Wr1_ubi, Wl1_ibu, bl1_ibu, Wr1_ibu,
           Wl2_ubi, bl2_ubi, Wr2_ubi, Wl2_ibu, bl2_ibu, Wr2_ibu,
           Wres_user, bres_user, Wres_item, bres_item):
    pad = ((0, NPAD - N), (0, 0))
    x_user_p = jnp.pad(x_user, pad)
    x_item_p = jnp.pad(x_item, pad)
    src_all = jnp.stack([edge_index_ubi[0], edge_index_ibu[0]]
                        ).reshape(2, NS, 2, NCHUNK2, CHUNK)
    dst_all = jnp.stack([edge_index_ubi[1], edge_index_ibu[1]]
                        ).reshape(2, NS, 2, NCHUNK2, CHUNK)
    dst_flat = dst_all.reshape(2, NS, NCHUNK, CHUNK)

    cnt = _seg_counts(dst_flat)
    agg1 = _seg_sum(jnp.stack([x_user_p, x_item_p]), src_all, dst_all)
    agg1_item, agg1_user = agg1[0], agg1[1]
    cnt_item, cnt_user = cnt[0], cnt[1]

    p_item, r2_item, rres_item = _layer1(
        agg1_item, cnt_item, x_item_p, Wl1_ubi, bl1_ubi, Wr1_ubi,
        Wl2_ibu, Wr2_ubi, Wres_item)
    p_user, r2_user, rres_user = _layer1(
        agg1_user, cnt_user, x_user_p, Wl1_ibu, bl1_ibu, Wr1_ibu,
        Wl2_ubi, Wr2_ibu, Wres_user)

    agg2 = _seg_sum(jnp.stack([p_user, p_item]), src_all, dst_all)
    agg2_item, agg2_user = agg2[0], agg2[1]

    out_item = _layer2(agg2_item, cnt_item, r2_item, rres_item,
                       bl2_ubi, bres_item)
    out_user = _layer2(agg2_user, cnt_user, r2_user, rres_user,
                       bl2_ibu, bres_user)
    return (out_user[:N], out_item[:N])


# speedup vs baseline: 8.0814x; 1.3190x over previous
import functools

import jax
import jax.numpy as jnp
from jax import lax
from jax.experimental import pallas as pl
from jax.experimental.pallas import tpu as pltpu
from jax.experimental.pallas import tpu_sc as plsc

N = 10000
NPAD = 10240
E = 320000
D = 128

NS = 16
EPT = E // NS
CHUNK = 80
NCHUNK = EPT // CHUNK
SCHUNK = 80
INNER = 25
NPASS = NCHUNK // INNER
RPT = NPAD // NS
ZROWS = 128


def _zero_fill(ref, rows, cols):
    zv = jnp.zeros((16,), jnp.float32)

    @pl.loop(0, rows)
    def _(i):
        for c in range(cols // 16):
            ref[i, pl.ds(c * 16, 16)] = zv


def _fill_ones(ref, rows, cols):
    ov = jnp.ones((16,), jnp.float32)

    @pl.loop(0, rows)
    def _(i):
        for c in range(cols // 16):
            ref[i, pl.ds(c * 16, 16)] = ov


def _make_seg_kernel():
    mesh = plsc.VectorSubcoreMesh(core_axis_name="c", subcore_axis_name="s",
                                  num_cores=2, num_subcores=NS)

    out_type = [jax.ShapeDtypeStruct((2, NPAD, D), jnp.float32)]
    scratch = [
        pltpu.VMEM((INNER, SCHUNK), jnp.int32),
        pltpu.VMEM((INNER, SCHUNK), jnp.int32),
        pltpu.VMEM((SCHUNK, D), jnp.float32),
        pltpu.VMEM((SCHUNK, D), jnp.float32),
        pltpu.VMEM((ZROWS, D), jnp.float32),
        pltpu.VMEM_SHARED((NPAD, D), jnp.float32),
        pltpu.SemaphoreType.DMA,
    ]

    def body(tab, src, dst, agg_out, src_v, dst_v, rows_a, rows_b, zrow_v,
             acc_sh, sem_a):
        cid = lax.axis_index("c")
        sid = lax.axis_index("s")

        _zero_fill(zrow_v, ZROWS, D)
        for k in range(RPT // ZROWS):
            pltpu.sync_copy(
                zrow_v, acc_sh.at[pl.ds(sid * RPT + k * ZROWS, ZROWS)])
        plsc.subcore_barrier()

        table = tab.at[cid]

        def wait_scat():
            pltpu.make_async_copy(
                rows_a, acc_sh.at[pl.ds(0, SCHUNK)], sem_a).wait()

        @pl.loop(0, NPASS)
        def _(p):
            pltpu.sync_copy(src.at[cid, sid, p], src_v)
            pltpu.sync_copy(dst.at[cid, sid, p], dst_v)

            pltpu.sync_copy(table.at[src_v.at[0]], rows_a)
            pltpu.async_copy(rows_a, acc_sh.at[dst_v.at[0]], sem_a,
                             add=True)

            @pl.loop(0, (INNER - 1) // 2)
            def _(q):
                j1 = 2 * q + 1
                pltpu.sync_copy(table.at[src_v.at[j1]], rows_b)
                wait_scat()
                pltpu.async_copy(rows_b, acc_sh.at[dst_v.at[j1]], sem_a,
                                 add=True)
                pltpu.sync_copy(table.at[src_v.at[j1 + 1]], rows_a)
                wait_scat()
                pltpu.async_copy(rows_a, acc_sh.at[dst_v.at[j1 + 1]], sem_a,
                                 add=True)

            wait_scat()

        plsc.subcore_barrier()
        pltpu.sync_copy(acc_sh.at[pl.ds(sid * RPT, RPT)],
                        agg_out.at[cid, pl.ds(sid * RPT, RPT)])

    return pl.kernel(
        body, out_type=out_type, mesh=mesh, scratch_types=scratch,
        compiler_params=pltpu.CompilerParams(use_tc_tiling_on_sc=False),
        name="seg_sum")


def _make_cnt_kernel():
    mesh = plsc.VectorSubcoreMesh(core_axis_name="c", subcore_axis_name="s",
                                  num_cores=2, num_subcores=NS)

    def body(dst, cnt_out, dst_v, ones_v, zcnt_v, cntacc_sh):
        cid = lax.axis_index("c")
        sid = lax.axis_index("s")

        _fill_ones(ones_v, CHUNK, 16)
        _zero_fill(zcnt_v, RPT, 16)
        pltpu.sync_copy(zcnt_v, cntacc_sh.at[pl.ds(sid * RPT, RPT)])
        plsc.subcore_barrier()

        pltpu.sync_copy(dst.at[cid, sid], dst_v)

        @pl.loop(0, NCHUNK)
        def _(j):
            pltpu.sync_copy(ones_v, cntacc_sh.at[dst_v.at[j]], add=True)

        plsc.subcore_barrier()
        pltpu.sync_copy(cntacc_sh.at[pl.ds(sid * RPT, RPT)],
                        cnt_out.at[cid, pl.ds(sid * RPT, RPT)])

    return pl.kernel(
        body,
        out_type=[jax.ShapeDtypeStruct((2, NPAD, 16), jnp.float32)],
        mesh=mesh,
        scratch_types=[
            pltpu.VMEM((NCHUNK, CHUNK), jnp.int32),
            pltpu.VMEM((CHUNK, 16), jnp.float32),
            pltpu.VMEM((RPT, 16), jnp.float32),
            pltpu.VMEM_SHARED((NPAD, 16), jnp.float32),
        ],
        compiler_params=pltpu.CompilerParams(use_tc_tiling_on_sc=False),
        name="seg_cnt")


_make_seg_kernel = functools.lru_cache(maxsize=None)(_make_seg_kernel)
_make_cnt_kernel = functools.lru_cache(maxsize=None)(_make_cnt_kernel)


def _seg_sum(tab, src, dst):
    (agg,) = _make_seg_kernel()(tab, src, dst)
    return agg


def _seg_counts(dst):
    (cnt,) = _make_cnt_kernel()(dst)
    return cnt


BLK = 1024


def _layer1_body(agg, cnt, x, Wl1, bl1, Wr1, Wl2x, Wr2x, Wres,
                 p, r2, rres):
    c = jnp.maximum(cnt[:, 0:1], 1.0)
    mean = agg[:] / c
    dn = (((1,), (1,)), ((), ()))
    t = (lax.dot_general(mean, Wl1[:], dn, preferred_element_type=jnp.float32)
         + bl1[:]
         + lax.dot_general(x[:], Wr1[:], dn,
                           preferred_element_type=jnp.float32))
    nrm = jnp.maximum(jnp.sqrt(jnp.sum(t * t, axis=1, keepdims=True)), 1e-12)
    x1 = jnp.maximum(t / nrm, 0.0)
    p[:] = lax.dot_general(x1, Wl2x[:], dn, preferred_element_type=jnp.float32)
    r2[:] = lax.dot_general(x1, Wr2x[:], dn,
                            preferred_element_type=jnp.float32)
    rres[:] = lax.dot_general(x1, Wres[:], dn,
                              preferred_element_type=jnp.float32)


def _layer1(agg, cnt, x, Wl1, bl1, Wr1, Wl2x, Wr2x, Wres):
    grid = NPAD // BLK
    return pl.pallas_call(
        _layer1_body,
        grid=grid,
        in_specs=[
            pl.BlockSpec((BLK, D), lambda i: (i, 0)),
            pl.BlockSpec((BLK, 16), lambda i: (i, 0)),
            pl.BlockSpec((BLK, D), lambda i: (i, 0)),
            pl.BlockSpec((2 * D, D), lambda i: (0, 0)),
            pl.BlockSpec((1, 2 * D), lambda i: (0, 0)),
            pl.BlockSpec((2 * D, D), lambda i: (0, 0)),
            pl.BlockSpec((D, 2 * D), lambda i: (0, 0)),
            pl.BlockSpec((D, 2 * D), lambda i: (0, 0)),
            pl.BlockSpec((D, 2 * D), lambda i: (0, 0)),
        ],
        out_specs=[
            pl.BlockSpec((BLK, D), lambda i: (i, 0)),
            pl.BlockSpec((BLK, D), lambda i: (i, 0)),
            pl.BlockSpec((BLK, D), lambda i: (i, 0)),
        ],
        out_shape=[jax.ShapeDtypeStruct((NPAD, D), jnp.float32)] * 3,
        name="sage_layer1",
    )(agg, cnt, x, Wl1, bl1.reshape(1, -1), Wr1, Wl2x, Wr2x, Wres)


def _layer2_body(agg2, cnt, r2, rres, bl2, bres, out):
    c = jnp.maximum(cnt[:, 0:1], 1.0)
    t = agg2[:] / c + bl2[:] + r2[:]
    nrm = jnp.maximum(jnp.sqrt(jnp.sum(t * t, axis=1, keepdims=True)), 1e-12)
    out[:] = t / nrm + rres[:] + bres[:]


def _layer2(agg2, cnt, r2, rres, bl2, bres):
    grid = NPAD // BLK
    return pl.pallas_call(
        _layer2_body,
        grid=grid,
        in_specs=[
            pl.BlockSpec((BLK, D), lambda i: (i, 0)),
            pl.BlockSpec((BLK, 16), lambda i: (i, 0)),
            pl.BlockSpec((BLK, D), lambda i: (i, 0)),
            pl.BlockSpec((BLK, D), lambda i: (i, 0)),
            pl.BlockSpec((1, D), lambda i: (0, 0)),
            pl.BlockSpec((1, D), lambda i: (0, 0)),
        ],
        out_specs=pl.BlockSpec((BLK, D), lambda i: (i, 0)),
        out_shape=jax.ShapeDtypeStruct((NPAD, D), jnp.float32),
        name="sage_layer2",
    )(agg2, cnt, r2, rres, bl2.reshape(1, -1), bres.reshape(1, -1))


def kernel(x_user, x_item, edge_index_ubi, edge_index_ibu,
           Wl1_ubi, bl1_ubi, Wr1_ubi, Wl1_ibu, bl1_ibu, Wr1_ibu,
           Wl2_ubi, bl2_ubi, Wr2_ubi, Wl2_ibu, bl2_ibu, Wr2_ibu,
           Wres_user, bres_user, Wres_item, bres_item):
    pad = ((0, NPAD - N), (0, 0))
    x_user_p = jnp.pad(x_user, pad)
    x_item_p = jnp.pad(x_item, pad)
    src_all = jnp.stack([edge_index_ubi[0], edge_index_ibu[0]]
                        ).reshape(2, NS, NPASS, INNER, SCHUNK)
    dst_all = jnp.stack([edge_index_ubi[1], edge_index_ibu[1]]
                        ).reshape(2, NS, NPASS, INNER, SCHUNK)
    dst_flat = dst_all.reshape(2, NS, NCHUNK, CHUNK)

    cnt = _seg_counts(dst_flat)
    agg1 = _seg_sum(jnp.stack([x_user_p, x_item_p]), src_all, dst_all)
    agg1_item, agg1_user = agg1[0], agg1[1]
    cnt_item, cnt_user = cnt[0], cnt[1]

    p_item, r2_item, rres_item = _layer1(
        agg1_item, cnt_item, x_item_p, Wl1_ubi, bl1_ubi, Wr1_ubi,
        Wl2_ibu, Wr2_ubi, Wres_item)
    p_user, r2_user, rres_user = _layer1(
        agg1_user, cnt_user, x_user_p, Wl1_ibu, bl1_ibu, Wr1_ibu,
        Wl2_ubi, Wr2_ibu, Wres_user)

    agg2 = _seg_sum(jnp.stack([p_user, p_item]), src_all, dst_all)
    agg2_item, agg2_user = agg2[0], agg2[1]

    out_item = _layer2(agg2_item, cnt_item, r2_item, rres_item,
                       bl2_ubi, bres_item)
    out_user = _layer2(agg2_user, cnt_user, r2_user, rres_user,
                       bl2_ibu, bres_user)
    return (out_user[:N], out_item[:N])


# speedup vs baseline: 9.8313x; 1.2165x over previous
import functools

import jax
import jax.numpy as jnp
from jax import lax
from jax.experimental import pallas as pl
from jax.experimental.pallas import tpu as pltpu
from jax.experimental.pallas import tpu_sc as plsc

N = 10000
NPAD = 10240
E = 320000
D = 128

NS = 16
EPT = E // NS
CHUNK = 80
NCHUNK = EPT // CHUNK
SCHUNK = 80
EPTP = EPT
INNER = 50
NPASS = EPTP // (INNER * SCHUNK)
RPT = NPAD // NS
ZROWS = 128


def _zero_fill(ref, rows, cols):
    zv = jnp.zeros((16,), jnp.float32)

    @pl.loop(0, rows)
    def _(i):
        for c in range(cols // 16):
            ref[i, pl.ds(c * 16, 16)] = zv


def _fill_ones(ref, rows, cols):
    ov = jnp.ones((16,), jnp.float32)

    @pl.loop(0, rows)
    def _(i):
        for c in range(cols // 16):
            ref[i, pl.ds(c * 16, 16)] = ov


def _make_seg_kernel():
    mesh = plsc.VectorSubcoreMesh(core_axis_name="c", subcore_axis_name="s",
                                  num_cores=2, num_subcores=NS)

    out_type = [jax.ShapeDtypeStruct((2, NPAD, D), jnp.float32)]
    scratch = [
        pltpu.VMEM((INNER, SCHUNK), jnp.int32),
        pltpu.VMEM((INNER, SCHUNK), jnp.int32),
        pltpu.VMEM((SCHUNK, D), jnp.float32),
        pltpu.VMEM((SCHUNK, D), jnp.float32),
        pltpu.VMEM((ZROWS, D), jnp.float32),
        pltpu.VMEM_SHARED((NPAD, D), jnp.float32),
        pltpu.SemaphoreType.DMA,
        pltpu.SemaphoreType.DMA,
    ]

    def body(tab, src, dst, agg_out, src_v, dst_v, rows_a, rows_b, zrow_v,
             acc_sh, sem_a, sem_g):
        cid = lax.axis_index("c")
        sid = lax.axis_index("s")

        _zero_fill(zrow_v, ZROWS, D)
        for k in range(RPT // ZROWS):
            pltpu.sync_copy(
                zrow_v, acc_sh.at[pl.ds(sid * RPT + k * ZROWS, ZROWS)])
        plsc.subcore_barrier()

        table = tab.at[cid]

        def wait_scat():
            pltpu.make_async_copy(
                rows_a, acc_sh.at[pl.ds(0, SCHUNK)], sem_a).wait()

        def wait_gath():
            pltpu.make_async_copy(
                table.at[pl.ds(0, SCHUNK)], rows_a, sem_g).wait()

        @pl.loop(0, NPASS)
        def _(p):
            pltpu.sync_copy(src.at[cid, sid, p], src_v)
            pltpu.sync_copy(dst.at[cid, sid, p], dst_v)

            pltpu.async_copy(table.at[src_v.at[0]], rows_a, sem_g)
            pltpu.async_copy(table.at[src_v.at[1]], rows_b, sem_g)

            @pl.loop(0, INNER // 2)
            def _(q):
                j0 = 2 * q
                wait_gath()
                pltpu.async_copy(rows_a, acc_sh.at[dst_v.at[j0]], sem_a,
                                 add=True)
                wait_gath()
                pltpu.async_copy(rows_b, acc_sh.at[dst_v.at[j0 + 1]], sem_a,
                                 add=True)
                j2 = jnp.minimum(j0 + 2, INNER - 1)
                j3 = jnp.minimum(j0 + 3, INNER - 1)
                wait_scat()
                pltpu.async_copy(table.at[src_v.at[j2]], rows_a, sem_g)
                wait_scat()
                pltpu.async_copy(table.at[src_v.at[j3]], rows_b, sem_g)

            wait_gath()
            wait_gath()

        plsc.subcore_barrier()
        pltpu.sync_copy(acc_sh.at[pl.ds(sid * RPT, RPT)],
                        agg_out.at[cid, pl.ds(sid * RPT, RPT)])

    return pl.kernel(
        body, out_type=out_type, mesh=mesh, scratch_types=scratch,
        compiler_params=pltpu.CompilerParams(use_tc_tiling_on_sc=False),
        name="seg_sum")


def _make_cnt_kernel():
    mesh = plsc.VectorSubcoreMesh(core_axis_name="c", subcore_axis_name="s",
                                  num_cores=2, num_subcores=NS)

    def body(dst, cnt_out, dst_v, ones_v, zcnt_v, cntacc_sh):
        cid = lax.axis_index("c")
        sid = lax.axis_index("s")

        _fill_ones(ones_v, CHUNK, 16)
        _zero_fill(zcnt_v, RPT, 16)
        pltpu.sync_copy(zcnt_v, cntacc_sh.at[pl.ds(sid * RPT, RPT)])
        plsc.subcore_barrier()

        pltpu.sync_copy(dst.at[cid, sid], dst_v)

        @pl.loop(0, NCHUNK)
        def _(j):
            pltpu.sync_copy(ones_v, cntacc_sh.at[dst_v.at[j]], add=True)

        plsc.subcore_barrier()
        pltpu.sync_copy(cntacc_sh.at[pl.ds(sid * RPT, RPT)],
                        cnt_out.at[cid, pl.ds(sid * RPT, RPT)])

    return pl.kernel(
        body,
        out_type=[jax.ShapeDtypeStruct((2, NPAD, 16), jnp.float32)],
        mesh=mesh,
        scratch_types=[
            pltpu.VMEM((NCHUNK, CHUNK), jnp.int32),
            pltpu.VMEM((CHUNK, 16), jnp.float32),
            pltpu.VMEM((RPT, 16), jnp.float32),
            pltpu.VMEM_SHARED((NPAD, 16), jnp.float32),
        ],
        compiler_params=pltpu.CompilerParams(use_tc_tiling_on_sc=False),
        name="seg_cnt")


_make_seg_kernel = functools.lru_cache(maxsize=None)(_make_seg_kernel)
_make_cnt_kernel = functools.lru_cache(maxsize=None)(_make_cnt_kernel)


def _seg_sum(tab, src, dst):
    (agg,) = _make_seg_kernel()(tab, src, dst)
    return agg


def _seg_counts(dst):
    (cnt,) = _make_cnt_kernel()(dst)
    return cnt


BLK = 1024


def _layer1p_body(agg, cnt, x, Wl1, bl1, Wr1, Wl2x, x1o, p):
    c = jnp.maximum(cnt[:, 0:1], 1.0)
    mean = agg[:] / c
    dn = (((1,), (1,)), ((), ()))
    t = (lax.dot_general(mean, Wl1[:], dn, preferred_element_type=jnp.float32)
         + bl1[:]
         + lax.dot_general(x[:], Wr1[:], dn,
                           preferred_element_type=jnp.float32))
    nrm = jnp.maximum(jnp.sqrt(jnp.sum(t * t, axis=1, keepdims=True)), 1e-12)
    x1 = jnp.maximum(t / nrm, 0.0)
    x1o[:] = x1
    p[:] = lax.dot_general(x1, Wl2x[:], dn, preferred_element_type=jnp.float32)


def _layer1p(agg, cnt, x, Wl1, bl1, Wr1, Wl2x):
    grid = NPAD // BLK
    return pl.pallas_call(
        _layer1p_body,
        grid=grid,
        in_specs=[
            pl.BlockSpec((BLK, D), lambda i: (i, 0)),
            pl.BlockSpec((BLK, 16), lambda i: (i, 0)),
            pl.BlockSpec((BLK, D), lambda i: (i, 0)),
            pl.BlockSpec((2 * D, D), lambda i: (0, 0)),
            pl.BlockSpec((1, 2 * D), lambda i: (0, 0)),
            pl.BlockSpec((2 * D, D), lambda i: (0, 0)),
            pl.BlockSpec((D, 2 * D), lambda i: (0, 0)),
        ],
        out_specs=[
            pl.BlockSpec((BLK, 2 * D), lambda i: (i, 0)),
            pl.BlockSpec((BLK, D), lambda i: (i, 0)),
        ],
        out_shape=[jax.ShapeDtypeStruct((NPAD, 2 * D), jnp.float32),
                   jax.ShapeDtypeStruct((NPAD, D), jnp.float32)],
        name="sage_layer1p",
    )(agg, cnt, x, Wl1, bl1.reshape(1, -1), Wr1, Wl2x)


def _layer1r_body(x1, Wr2x, Wres, r2, rres):
    dn = (((1,), (1,)), ((), ()))
    r2[:] = lax.dot_general(x1[:], Wr2x[:], dn,
                            preferred_element_type=jnp.float32)
    rres[:] = lax.dot_general(x1[:], Wres[:], dn,
                              preferred_element_type=jnp.float32)


def _layer1r(x1, Wr2x, Wres):
    grid = NPAD // BLK
    return pl.pallas_call(
        _layer1r_body,
        grid=grid,
        in_specs=[
            pl.BlockSpec((BLK, 2 * D), lambda i: (i, 0)),
            pl.BlockSpec((D, 2 * D), lambda i: (0, 0)),
            pl.BlockSpec((D, 2 * D), lambda i: (0, 0)),
        ],
        out_specs=[
            pl.BlockSpec((BLK, D), lambda i: (i, 0)),
            pl.BlockSpec((BLK, D), lambda i: (i, 0)),
        ],
        out_shape=[jax.ShapeDtypeStruct((NPAD, D), jnp.float32)] * 2,
        name="sage_layer1r",
    )(x1, Wr2x, Wres)


def _layer2_body(agg2, cnt, r2, rres, bl2, bres, out):
    c = jnp.maximum(cnt[:, 0:1], 1.0)
    t = agg2[:] / c + bl2[:] + r2[:]
    nrm = jnp.maximum(jnp.sqrt(jnp.sum(t * t, axis=1, keepdims=True)), 1e-12)
    out[:] = t / nrm + rres[:] + bres[:]


def _layer2(agg2, cnt, r2, rres, bl2, bres):
    grid = NPAD // BLK
    return pl.pallas_call(
        _layer2_body,
        grid=grid,
        in_specs=[
            pl.BlockSpec((BLK, D), lambda i: (i, 0)),
            pl.BlockSpec((BLK, 16), lambda i: (i, 0)),
            pl.BlockSpec((BLK, D), lambda i: (i, 0)),
            pl.BlockSpec((BLK, D), lambda i: (i, 0)),
            pl.BlockSpec((1, D), lambda i: (0, 0)),
            pl.BlockSpec((1, D), lambda i: (0, 0)),
        ],
        out_specs=pl.BlockSpec((BLK, D), lambda i: (i, 0)),
        out_shape=jax.ShapeDtypeStruct((NPAD, D), jnp.float32),
        name="sage_layer2",
    )(agg2, cnt, r2, rres, bl2.reshape(1, -1), bres.reshape(1, -1))


def kernel(x_user, x_item, edge_index_ubi, edge_index_ibu,
           Wl1_ubi, bl1_ubi, Wr1_ubi, Wl1_ibu, bl1_ibu, Wr1_ibu,
           Wl2_ubi, bl2_ubi, Wr2_ubi, Wl2_ibu, bl2_ibu, Wr2_ibu,
           Wres_user, bres_user, Wres_item, bres_item):
    pad = ((0, NPAD - N), (0, 0))
    x_user_p = jnp.pad(x_user, pad)
    x_item_p = jnp.pad(x_item, pad)
    src_r = jnp.stack([edge_index_ubi[0], edge_index_ibu[0]]
                      ).reshape(2, NS, EPT)
    dst_r = jnp.stack([edge_index_ubi[1], edge_index_ibu[1]]
                      ).reshape(2, NS, EPT)
    epad = ((0, 0), (0, 0), (0, EPTP - EPT))
    src_all = jnp.pad(src_r, epad).reshape(2, NS, NPASS, INNER, SCHUNK)
    dst_all = jnp.pad(dst_r, epad, constant_values=NPAD - 1
                      ).reshape(2, NS, NPASS, INNER, SCHUNK)
    dst_flat = dst_r.reshape(2, NS, NCHUNK, CHUNK)

    cnt = _seg_counts(dst_flat)
    agg1 = _seg_sum(jnp.stack([x_user_p, x_item_p]), src_all, dst_all)
    agg1_item, agg1_user = agg1[0], agg1[1]
    cnt_item, cnt_user = cnt[0], cnt[1]

    x1_item, p_item = _layer1p(
        agg1_item, cnt_item, x_item_p, Wl1_ubi, bl1_ubi, Wr1_ubi, Wl2_ibu)
    x1_user, p_user = _layer1p(
        agg1_user, cnt_user, x_user_p, Wl1_ibu, bl1_ibu, Wr1_ibu, Wl2_ubi)

    agg2 = _seg_sum(jnp.stack([p_user, p_item]), src_all, dst_all)
    r2_item, rres_item = _layer1r(x1_item, Wr2_ubi, Wres_item)
    r2_user, rres_user = _layer1r(x1_user, Wr2_ibu, Wres_user)
    agg2_item, agg2_user = agg2[0], agg2[1]

    out_item = _layer2(agg2_item, cnt_item, r2_item, rres_item,
                       bl2_ubi, bres_item)
    out_user = _layer2(agg2_user, cnt_user, r2_user, rres_user,
                       bl2_ibu, bres_user)
    return (out_user[:N], out_item[:N])


# speedup vs baseline: 10.2987x; 1.0475x over previous
import functools

import jax
import jax.numpy as jnp
from jax import lax
from jax.experimental import pallas as pl
from jax.experimental.pallas import tpu as pltpu
from jax.experimental.pallas import tpu_sc as plsc

N = 10000
NPAD = 10240
E = 320000
D = 128

NS = 16
EPT = E // NS
CHUNK = 80
NCHUNK = EPT // CHUNK
SCHUNK = 80
EPTP = EPT
INNER = 50
NPASS = EPTP // (INNER * SCHUNK)
RPT = NPAD // NS
ZROWS = 128


def _zero_fill(ref, rows, cols):
    zv = jnp.zeros((16,), jnp.float32)

    @pl.loop(0, rows)
    def _(i):
        for c in range(cols // 16):
            ref[i, pl.ds(c * 16, 16)] = zv


def _fill_ones(ref, rows, cols):
    ov = jnp.ones((16,), jnp.float32)

    @pl.loop(0, rows)
    def _(i):
        for c in range(cols // 16):
            ref[i, pl.ds(c * 16, 16)] = ov


def _make_seg_kernel():
    mesh = plsc.VectorSubcoreMesh(core_axis_name="c", subcore_axis_name="s",
                                  num_cores=2, num_subcores=NS)

    out_type = [jax.ShapeDtypeStruct((2, NPAD, D), jnp.float32)]
    scratch = [
        pltpu.VMEM((INNER, SCHUNK), jnp.int32),
        pltpu.VMEM((INNER, SCHUNK), jnp.int32),
        pltpu.VMEM((SCHUNK, D), jnp.float32),
        pltpu.VMEM((SCHUNK, D), jnp.float32),
        pltpu.VMEM((ZROWS, D), jnp.float32),
        pltpu.VMEM_SHARED((NPAD, D), jnp.float32),
        pltpu.SemaphoreType.DMA,
        pltpu.SemaphoreType.DMA,
    ]

    def body(tab, src, dst, agg_out, src_v, dst_v, rows_a, rows_b, zrow_v,
             acc_sh, sem_a, sem_g):
        cid = lax.axis_index("c")
        sid = lax.axis_index("s")

        _zero_fill(zrow_v, ZROWS, D)
        for k in range(RPT // ZROWS):
            pltpu.sync_copy(
                zrow_v, acc_sh.at[pl.ds(sid * RPT + k * ZROWS, ZROWS)])
        plsc.subcore_barrier()

        table = tab.at[cid]

        def wait_scat():
            pltpu.make_async_copy(
                rows_a, acc_sh.at[pl.ds(0, SCHUNK)], sem_a).wait()

        def wait_gath():
            pltpu.make_async_copy(
                table.at[pl.ds(0, SCHUNK)], rows_a, sem_g).wait()

        @pl.loop(0, NPASS)
        def _(p):
            pltpu.sync_copy(src.at[cid, sid, p], src_v)
            pltpu.sync_copy(dst.at[cid, sid, p], dst_v)

            pltpu.async_copy(table.at[src_v.at[0]], rows_a, sem_g)
            pltpu.async_copy(table.at[src_v.at[1]], rows_b, sem_g)

            @pl.loop(0, INNER // 2)
            def _(q):
                j0 = 2 * q
                wait_gath()
                pltpu.async_copy(rows_a, acc_sh.at[dst_v.at[j0]], sem_a,
                                 add=True)
                wait_gath()
                pltpu.async_copy(rows_b, acc_sh.at[dst_v.at[j0 + 1]], sem_a,
                                 add=True)
                j2 = jnp.minimum(j0 + 2, INNER - 1)
                j3 = jnp.minimum(j0 + 3, INNER - 1)
                wait_scat()
                pltpu.async_copy(table.at[src_v.at[j2]], rows_a, sem_g)
                wait_scat()
                pltpu.async_copy(table.at[src_v.at[j3]], rows_b, sem_g)

            wait_gath()
            wait_gath()

        plsc.subcore_barrier()
        pltpu.sync_copy(acc_sh.at[pl.ds(sid * RPT, RPT)],
                        agg_out.at[cid, pl.ds(sid * RPT, RPT)])

    return pl.kernel(
        body, out_type=out_type, mesh=mesh, scratch_types=scratch,
        compiler_params=pltpu.CompilerParams(use_tc_tiling_on_sc=False),
        name="seg_sum")


def _make_cnt_kernel():
    mesh = plsc.VectorSubcoreMesh(core_axis_name="c", subcore_axis_name="s",
                                  num_cores=2, num_subcores=NS)

    def body(dst, cnt_out, dst_v, ones_v, zcnt_v, cntacc_sh):
        cid = lax.axis_index("c")
        sid = lax.axis_index("s")

        _fill_ones(ones_v, CHUNK, 16)
        _zero_fill(zcnt_v, RPT, 16)
        pltpu.sync_copy(zcnt_v, cntacc_sh.at[pl.ds(sid * RPT, RPT)])
        plsc.subcore_barrier()

        pltpu.sync_copy(dst.at[cid, sid], dst_v)

        @pl.loop(0, NCHUNK)
        def _(j):
            pltpu.sync_copy(ones_v, cntacc_sh.at[dst_v.at[j]], add=True)

        plsc.subcore_barrier()
        pltpu.sync_copy(cntacc_sh.at[pl.ds(sid * RPT, RPT)],
                        cnt_out.at[cid, pl.ds(sid * RPT, RPT)])

    return pl.kernel(
        body,
        out_type=[jax.ShapeDtypeStruct((2, NPAD, 16), jnp.float32)],
        mesh=mesh,
        scratch_types=[
            pltpu.VMEM((NCHUNK, CHUNK), jnp.int32),
            pltpu.VMEM((CHUNK, 16), jnp.float32),
            pltpu.VMEM((RPT, 16), jnp.float32),
            pltpu.VMEM_SHARED((NPAD, 16), jnp.float32),
        ],
        compiler_params=pltpu.CompilerParams(use_tc_tiling_on_sc=False),
        name="seg_cnt")


_make_seg_kernel = functools.lru_cache(maxsize=None)(_make_seg_kernel)
_make_cnt_kernel = functools.lru_cache(maxsize=None)(_make_cnt_kernel)


def _seg_sum(tab, src, dst):
    (agg,) = _make_seg_kernel()(tab, src, dst)
    return agg


def _seg_counts(dst):
    (cnt,) = _make_cnt_kernel()(dst)
    return cnt


BLK = 1024


NT = NPAD // BLK


def _layer1_body(agg, cnt, x, Wl1, bl1, Wr1, Wl2x, Wr2x, Wres,
                 p, r2, rres):
    c = jnp.maximum(cnt[0][:, 0:1], 1.0)
    mean = agg[0] / c
    dn = (((1,), (1,)), ((), ()))
    t = (lax.dot_general(mean, Wl1[0], dn, preferred_element_type=jnp.float32)
         + bl1[0]
         + lax.dot_general(x[0], Wr1[0], dn,
                           preferred_element_type=jnp.float32))
    nrm = jnp.maximum(jnp.sqrt(jnp.sum(t * t, axis=1, keepdims=True)), 1e-12)
    x1 = jnp.maximum(t / nrm, 0.0)
    p[0] = lax.dot_general(x1, Wl2x[0], dn, preferred_element_type=jnp.float32)
    r2[0] = lax.dot_general(x1, Wr2x[0], dn,
                            preferred_element_type=jnp.float32)
    rres[0] = lax.dot_general(x1, Wres[0], dn,
                              preferred_element_type=jnp.float32)


def _layer1(agg, cnt, xs, Wl1, bl1, Wr1, Wl2x, Wr2x, Wres):
    row = lambda i: (i // NT, i % NT, 0)
    rrow = lambda i: (1 - i // NT, i % NT, 0)
    wgt = lambda i: (i // NT, 0, 0)
    return pl.pallas_call(
        _layer1_body,
        grid=2 * NT,
        in_specs=[
            pl.BlockSpec((1, BLK, D), row),
            pl.BlockSpec((1, BLK, 16), row),
            pl.BlockSpec((1, BLK, D), rrow),
            pl.BlockSpec((1, 2 * D, D), wgt),
            pl.BlockSpec((1, 1, 2 * D), wgt),
            pl.BlockSpec((1, 2 * D, D), wgt),
            pl.BlockSpec((1, D, 2 * D), wgt),
            pl.BlockSpec((1, D, 2 * D), wgt),
            pl.BlockSpec((1, D, 2 * D), wgt),
        ],
        out_specs=[
            pl.BlockSpec((1, BLK, D), rrow),
            pl.BlockSpec((1, BLK, D), row),
            pl.BlockSpec((1, BLK, D), row),
        ],
        out_shape=[jax.ShapeDtypeStruct((2, NPAD, D), jnp.float32)] * 3,
        name="sage_layer1",
    )(agg, cnt, xs, Wl1, bl1, Wr1, Wl2x, Wr2x, Wres)


def _layer2_body(agg2, cnt, r2, rres, bl2, bres, out):
    c = jnp.maximum(cnt[0][:, 0:1], 1.0)
    t = agg2[0] / c + bl2[0] + r2[0]
    nrm = jnp.maximum(jnp.sqrt(jnp.sum(t * t, axis=1, keepdims=True)), 1e-12)
    out[0] = t / nrm + rres[0] + bres[0]


def _layer2(agg2, cnt, r2, rres, bl2, bres):
    row = lambda i: (i // NT, i % NT, 0)
    wgt = lambda i: (i // NT, 0, 0)
    return pl.pallas_call(
        _layer2_body,
        grid=2 * NT,
        in_specs=[
            pl.BlockSpec((1, BLK, D), row),
            pl.BlockSpec((1, BLK, 16), row),
            pl.BlockSpec((1, BLK, D), row),
            pl.BlockSpec((1, BLK, D), row),
            pl.BlockSpec((1, 1, D), wgt),
            pl.BlockSpec((1, 1, D), wgt),
        ],
        out_specs=pl.BlockSpec((1, BLK, D), row),
        out_shape=jax.ShapeDtypeStruct((2, NPAD, D), jnp.float32),
        name="sage_layer2",
    )(agg2, cnt, r2, rres, bl2, bres)


def kernel(x_user, x_item, edge_index_ubi, edge_index_ibu,
           Wl1_ubi, bl1_ubi, Wr1_ubi, Wl1_ibu, bl1_ibu, Wr1_ibu,
           Wl2_ubi, bl2_ubi, Wr2_ubi, Wl2_ibu, bl2_ibu, Wr2_ibu,
           Wres_user, bres_user, Wres_item, bres_item):
    pad = ((0, NPAD - N), (0, 0))
    x_user_p = jnp.pad(x_user, pad)
    x_item_p = jnp.pad(x_item, pad)
    src_r = jnp.stack([edge_index_ubi[0], edge_index_ibu[0]]
                      ).reshape(2, NS, EPT)
    dst_r = jnp.stack([edge_index_ubi[1], edge_index_ibu[1]]
                      ).reshape(2, NS, EPT)
    epad = ((0, 0), (0, 0), (0, EPTP - EPT))
    src_all = jnp.pad(src_r, epad).reshape(2, NS, NPASS, INNER, SCHUNK)
    dst_all = jnp.pad(dst_r, epad, constant_values=NPAD - 1
                      ).reshape(2, NS, NPASS, INNER, SCHUNK)
    dst_flat = dst_r.reshape(2, NS, NCHUNK, CHUNK)

    xs = jnp.stack([x_user_p, x_item_p])
    cnt = _seg_counts(dst_flat)
    agg1 = _seg_sum(xs, src_all, dst_all)

    p_s, r2_s, rres_s = _layer1(
        agg1, cnt, xs,
        jnp.stack([Wl1_ubi, Wl1_ibu]),
        jnp.stack([bl1_ubi, bl1_ibu]).reshape(2, 1, 2 * D),
        jnp.stack([Wr1_ubi, Wr1_ibu]),
        jnp.stack([Wl2_ibu, Wl2_ubi]),
        jnp.stack([Wr2_ubi, Wr2_ibu]),
        jnp.stack([Wres_item, Wres_user]))

    agg2 = _seg_sum(p_s, src_all, dst_all)

    out_s = _layer2(agg2, cnt, r2_s, rres_s,
                    jnp.stack([bl2_ubi, bl2_ibu]).reshape(2, 1, D),
                    jnp.stack([bres_item, bres_user]).reshape(2, 1, D))
    return (out_s[1, :N], out_s[0, :N])
